# Initial kernel scaffold; baseline (speedup 1.0000x reference)
import functools

import jax
import jax.numpy as jnp
from jax import lax
from jax.experimental import pallas as pl
from jax.experimental.pallas import tpu as pltpu
from jax.experimental.pallas import tpu_sc as plsc

_N = 10000
_E = 320000
_G = 64
_D = 128

_NC = 2
_NS = 16
_NW = _NC * _NS

_N_PAD = 10240
_IDXW = 128
_E_ROWS = 2560
_E_PAD = _E_ROWS * _IDXW
_IR_PER_TILE = _E_ROWS // _NW

_mesh = plsc.VectorSubcoreMesh(
    core_axis_name="c", subcore_axis_name="s", num_cores=_NC, num_subcores=_NS
)


@functools.partial(
    pl.kernel,
    out_type=jax.ShapeDtypeStruct((_NC, _N_PAD, 16), jnp.float32),
    mesh=_mesh,
    scratch_types=[
        pltpu.VMEM((_IR_PER_TILE, _IDXW), jnp.int32),
        pltpu.VMEM((_IDXW, 16), jnp.float32),
        pltpu.VMEM_SHARED((_N_PAD, 16), jnp.float32),
    ],
)
def _sc_deg(dst_hbm, ones_hbm, zeros_hbm, out_hbm, dst_v, ones_v, deg_sh):
    cid = lax.axis_index("c")
    sid = lax.axis_index("s")
    wid = cid * _NS + sid
    nz = _N_PAD // _NS
    pltpu.sync_copy(zeros_hbm.at[pl.ds(sid * nz, nz)], deg_sh.at[pl.ds(sid * nz, nz)])
    pltpu.sync_copy(ones_hbm, ones_v)
    pltpu.sync_copy(dst_hbm.at[pl.ds(wid * _IR_PER_TILE, _IR_PER_TILE)], dst_v)
    plsc.subcore_barrier()

    def body(j, carry):
        pltpu.sync_copy(ones_v, deg_sh.at[dst_v.at[j]], add=True)
        return carry

    lax.fori_loop(0, _IR_PER_TILE, body, 0)
    plsc.subcore_barrier()
    pltpu.sync_copy(
        deg_sh.at[pl.ds(sid * nz, nz)], out_hbm.at[cid, pl.ds(sid * nz, nz)]
    )


@functools.partial(
    pl.kernel,
    out_type=jax.ShapeDtypeStruct((_NC, _N_PAD, _D), jnp.float32),
    mesh=_mesh,
    scratch_types=[
        pltpu.VMEM((_IR_PER_TILE, _IDXW), jnp.int32),
        pltpu.VMEM((_IR_PER_TILE, _IDXW), jnp.int32),
        pltpu.VMEM((_IDXW, _D), jnp.float32),
        pltpu.VMEM_SHARED((_N_PAD, _D), jnp.float32),
        pltpu.SemaphoreType.DMA,
    ],
)
def _sc_edge(y_hbm, src_hbm, dst_hbm, zeros_hbm, out_hbm, src_v, dst_v, rows_v,
             acc_sh, sem):
    cid = lax.axis_index("c")
    sid = lax.axis_index("s")
    wid = cid * _NS + sid
    nz = _N_PAD // _NS
    pltpu.sync_copy(zeros_hbm.at[pl.ds(sid * nz, nz)], acc_sh.at[pl.ds(sid * nz, nz)])
    pltpu.sync_copy(src_hbm.at[pl.ds(wid * _IR_PER_TILE, _IR_PER_TILE)], src_v)
    pltpu.sync_copy(dst_hbm.at[pl.ds(wid * _IR_PER_TILE, _IR_PER_TILE)], dst_v)
    plsc.subcore_barrier()

    def body(j, carry):
        pltpu.async_copy(y_hbm.at[src_v.at[j]], rows_v, sem).wait()
        pltpu.sync_copy(rows_v, acc_sh.at[dst_v.at[j]], add=True)
        return carry

    lax.fori_loop(0, _IR_PER_TILE, body, 0)
    plsc.subcore_barrier()
    pltpu.sync_copy(
        acc_sh.at[pl.ds(sid * nz, nz)], out_hbm.at[cid, pl.ds(sid * nz, nz)]
    )


_BLK = 1000
_NBLK = _N // _BLK


def _dinv_of(deg_ref):
    deg = deg_ref[0, :, 0:1] + deg_ref[1, :, 0:1] + 1.0
    return lax.rsqrt(deg)


def _tc_first_body(x_ref, w_ref, deg_ref, y_ref):
    dinv = _dinv_of(deg_ref)
    y_ref[...] = (
        jnp.dot(x_ref[...], w_ref[...], preferred_element_type=jnp.float32) * dinv
    )


def _tc_first(x, W, deg2):
    return pl.pallas_call(
        _tc_first_body,
        grid=(_NBLK,),
        in_specs=[
            pl.BlockSpec((_BLK, _D), lambda i: (i, 0)),
            pl.BlockSpec((_D, _D), lambda i: (0, 0)),
            pl.BlockSpec((_NC, _BLK, 16), lambda i: (0, i, 0)),
        ],
        out_specs=pl.BlockSpec((_BLK, _D), lambda i: (i, 0)),
        out_shape=jax.ShapeDtypeStruct((_N, _D), jnp.float32),
    )(x, W, deg2)


def _tc_mid_body(acc_ref, y_ref, deg_ref, b_ref, w_ref, out_ref):
    dinv = _dinv_of(deg_ref)
    s = acc_ref[0] + acc_ref[1] + y_ref[...]
    h = jnp.maximum(dinv * s + b_ref[...], 0.0)
    out_ref[...] = (
        jnp.dot(h, w_ref[...], preferred_element_type=jnp.float32) * dinv
    )


def _tc_mid(acc, y, deg2, b, Wn):
    return pl.pallas_call(
        _tc_mid_body,
        grid=(_NBLK,),
        in_specs=[
            pl.BlockSpec((_NC, _BLK, _D), lambda i: (0, i, 0)),
            pl.BlockSpec((_BLK, _D), lambda i: (i, 0)),
            pl.BlockSpec((_NC, _BLK, 16), lambda i: (0, i, 0)),
            pl.BlockSpec((1, _D), lambda i: (0, 0)),
            pl.BlockSpec((_D, _D), lambda i: (0, 0)),
        ],
        out_specs=pl.BlockSpec((_BLK, _D), lambda i: (i, 0)),
        out_shape=jax.ShapeDtypeStruct((_N, _D), jnp.float32),
    )(acc, y, deg2, b, Wn)


def _tc_final_body(acc_ref, y_ref, deg_ref, b_ref, batch_ref, fw1_ref, fb1_ref,
                   fw2_ref, fb2_ref, out_ref, hg_acc, cnt_acc):
    i = pl.program_id(0)

    @pl.when(i == 0)
    def _():
        hg_acc[...] = jnp.zeros_like(hg_acc)
        cnt_acc[...] = jnp.zeros_like(cnt_acc)

    dinv = _dinv_of(deg_ref)
    s = acc_ref[0] + acc_ref[1] + y_ref[...]
    h = jnp.maximum(dinv * s + b_ref[...], 0.0)
    bvec = batch_ref[0, 0, :]
    onehot = (
        bvec[None, :] == lax.broadcasted_iota(jnp.int32, (_G, _BLK), 0)
    ).astype(jnp.float32)
    hg_acc[...] += jnp.dot(onehot, h, preferred_element_type=jnp.float32)
    cnt_acc[...] += jnp.dot(
        onehot, jnp.ones((_BLK, _D), jnp.float32), preferred_element_type=jnp.float32
    )

    @pl.when(i == _NBLK - 1)
    def _():
        hg = hg_acc[...] / jnp.maximum(cnt_acc[...], 1.0)
        a = jnp.maximum(
            jnp.dot(hg, fw1_ref[...], preferred_element_type=jnp.float32)
            + fb1_ref[...],
            0.0,
        )
        out_ref[...] = (
            jnp.dot(a, fw2_ref[...], preferred_element_type=jnp.float32)
            + fb2_ref[...]
        )


def _tc_final(acc, y, deg2, b, batch3, fW1, fb1, fW2, fb2):
    return pl.pallas_call(
        _tc_final_body,
        grid=(_NBLK,),
        in_specs=[
            pl.BlockSpec((_NC, _BLK, _D), lambda i: (0, i, 0)),
            pl.BlockSpec((_BLK, _D), lambda i: (i, 0)),
            pl.BlockSpec((_NC, _BLK, 16), lambda i: (0, i, 0)),
            pl.BlockSpec((1, _D), lambda i: (0, 0)),
            pl.BlockSpec((1, 1, _BLK), lambda i: (i, 0, 0)),
            pl.BlockSpec((_D, _G), lambda i: (0, 0)),
            pl.BlockSpec((1, _G), lambda i: (0, 0)),
            pl.BlockSpec((_G, 1), lambda i: (0, 0)),
            pl.BlockSpec((1, 1), lambda i: (0, 0)),
        ],
        out_specs=pl.BlockSpec((_G, 1), lambda i: (0, 0)),
        out_shape=jax.ShapeDtypeStruct((_G, 1), jnp.float32),
        scratch_shapes=[
            pltpu.VMEM((_G, _D), jnp.float32),
            pltpu.VMEM((_G, _D), jnp.float32),
        ],
    )(acc, y, deg2, b, batch3, fW1, fb1, fW2, fb2)


def kernel(x, edge_index, batch, W1, b1, W2, b2, W3, b3, fW1, fb1, fW2, fb2):
    src = edge_index[0].astype(jnp.int32)
    dst = edge_index[1].astype(jnp.int32)
    pad_src = jnp.zeros((_E_PAD - _E,), jnp.int32)
    pad_dst = jnp.full((_E_PAD - _E,), _N_PAD - 1, jnp.int32)
    src2d = jnp.concatenate([src, pad_src]).reshape(_E_ROWS, _IDXW)
    dst2d = jnp.concatenate([dst, pad_dst]).reshape(_E_ROWS, _IDXW)

    ones16 = jnp.ones((_IDXW, 16), jnp.float32)
    zeros16 = jnp.zeros((_N_PAD, 16), jnp.float32)
    zeros128 = jnp.zeros((_N_PAD, _D), jnp.float32)

    deg2 = _sc_deg(dst2d, ones16, zeros16)
    y1 = _tc_first(x, W1, deg2)
    acc1 = _sc_edge(y1, src2d, dst2d, zeros128)
    y2 = _tc_mid(acc1, y1, deg2, b1.reshape(1, _D), W2)
    acc2 = _sc_edge(y2, src2d, dst2d, zeros128)
    y3 = _tc_mid(acc2, y2, deg2, b2.reshape(1, _D), W3)
    acc3 = _sc_edge(y3, src2d, dst2d, zeros128)

    batch3 = batch.astype(jnp.int32).reshape(_NBLK, 1, _BLK)
    return _tc_final(
        acc3, y3, deg2, b3.reshape(1, _D), batch3,
        fW1, fb1.reshape(1, _G), fW2, fb2.reshape(1, 1),
    )


# speedup vs baseline: 5.5650x; 5.5650x over previous
import functools

import jax
import jax.numpy as jnp
from jax import lax
from jax.experimental import pallas as pl
from jax.experimental.pallas import tpu as pltpu
from jax.experimental.pallas import tpu_sc as plsc

_N = 10000
_E = 320000
_G = 64
_D = 128

_NC = 2
_NS = 16
_NW = _NC * _NS

_N_PAD = 10240
_IDXW = 128
_E_ROWS = 2560
_E_PAD = _E_ROWS * _IDXW
_IR_PER_TILE = _E_ROWS // _NW

_mesh = plsc.VectorSubcoreMesh(
    core_axis_name="c", subcore_axis_name="s", num_cores=_NC, num_subcores=_NS
)


@functools.partial(
    pl.kernel,
    out_type=jax.ShapeDtypeStruct((_NC, _N_PAD, _D), jnp.float32),
    mesh=_mesh,
    scratch_types=[
        pltpu.VMEM((_IDXW,), jnp.int32),
        pltpu.VMEM((_IDXW, _D), jnp.float32),
        pltpu.VMEM_SHARED((_N_PAD, _D), jnp.float32),
    ],
)
def _sc_deg(dst_hbm, ones_hbm, zeros_hbm, out_hbm, dstc_v, ones_v, deg_sh):
    cid = lax.axis_index("c")
    sid = lax.axis_index("s")
    wid = cid * _NS + sid
    nz = _N_PAD // _NS
    pltpu.sync_copy(zeros_hbm.at[pl.ds(sid * nz, nz)], deg_sh.at[pl.ds(sid * nz, nz)])
    pltpu.sync_copy(ones_hbm, ones_v)
    plsc.subcore_barrier()

    def body(j, carry):
        pltpu.sync_copy(dst_hbm.at[wid * _IR_PER_TILE + j], dstc_v)
        pltpu.sync_copy(ones_v, deg_sh.at[dstc_v], add=True)
        return carry

    lax.fori_loop(0, _IR_PER_TILE, body, 0)
    plsc.subcore_barrier()
    pltpu.sync_copy(
        deg_sh.at[pl.ds(sid * nz, nz)], out_hbm.at[cid, pl.ds(sid * nz, nz)]
    )


@functools.partial(
    pl.kernel,
    out_type=jax.ShapeDtypeStruct((_NC, _N_PAD, _D), jnp.float32),
    mesh=_mesh,
    scratch_types=[
        pltpu.VMEM((_IDXW,), jnp.int32),
        pltpu.VMEM((_IDXW,), jnp.int32),
        pltpu.VMEM((_IDXW, _D), jnp.float32),
        pltpu.VMEM_SHARED((_N_PAD, _D), jnp.float32),
        pltpu.SemaphoreType.DMA,
    ],
)
def _sc_edge(y_hbm, src_hbm, dst_hbm, zeros_hbm, out_hbm, srcc_v, dstc_v,
             rows_v, acc_sh, sem):
    cid = lax.axis_index("c")
    sid = lax.axis_index("s")
    wid = cid * _NS + sid
    nz = _N_PAD // _NS
    pltpu.sync_copy(zeros_hbm.at[pl.ds(sid * nz, nz)], acc_sh.at[pl.ds(sid * nz, nz)])
    plsc.subcore_barrier()

    def body(j, carry):
        pltpu.sync_copy(src_hbm.at[wid * _IR_PER_TILE + j], srcc_v)
        pltpu.sync_copy(dst_hbm.at[wid * _IR_PER_TILE + j], dstc_v)
        pltpu.async_copy(y_hbm.at[srcc_v], rows_v, sem).wait()
        pltpu.sync_copy(rows_v, acc_sh.at[dstc_v], add=True)
        return carry

    lax.fori_loop(0, _IR_PER_TILE, body, 0)
    plsc.subcore_barrier()
    pltpu.sync_copy(
        acc_sh.at[pl.ds(sid * nz, nz)], out_hbm.at[cid, pl.ds(sid * nz, nz)]
    )


_BLK = 1000
_NBLK = _N // _BLK


def _dinv_of(deg_ref):
    deg = deg_ref[0, :, 0:1] + deg_ref[1, :, 0:1] + 1.0
    return lax.rsqrt(deg)


def _tc_first_body(x_ref, w_ref, deg_ref, y_ref):
    dinv = _dinv_of(deg_ref)
    y_ref[...] = (
        jnp.dot(x_ref[...], w_ref[...], preferred_element_type=jnp.float32) * dinv
    )


def _tc_first(x, W, deg2):
    return pl.pallas_call(
        _tc_first_body,
        grid=(_NBLK,),
        in_specs=[
            pl.BlockSpec((_BLK, _D), lambda i: (i, 0)),
            pl.BlockSpec((_D, _D), lambda i: (0, 0)),
            pl.BlockSpec((_NC, _BLK, _D), lambda i: (0, i, 0)),
        ],
        out_specs=pl.BlockSpec((_BLK, _D), lambda i: (i, 0)),
        out_shape=jax.ShapeDtypeStruct((_N, _D), jnp.float32),
    )(x, W, deg2)


def _tc_mid_body(acc_ref, y_ref, deg_ref, b_ref, w_ref, out_ref):
    dinv = _dinv_of(deg_ref)
    s = acc_ref[0] + acc_ref[1] + y_ref[...]
    h = jnp.maximum(dinv * s + b_ref[...], 0.0)
    out_ref[...] = (
        jnp.dot(h, w_ref[...], preferred_element_type=jnp.float32) * dinv
    )


def _tc_mid(acc, y, deg2, b, Wn):
    return pl.pallas_call(
        _tc_mid_body,
        grid=(_NBLK,),
        in_specs=[
            pl.BlockSpec((_NC, _BLK, _D), lambda i: (0, i, 0)),
            pl.BlockSpec((_BLK, _D), lambda i: (i, 0)),
            pl.BlockSpec((_NC, _BLK, _D), lambda i: (0, i, 0)),
            pl.BlockSpec((1, _D), lambda i: (0, 0)),
            pl.BlockSpec((_D, _D), lambda i: (0, 0)),
        ],
        out_specs=pl.BlockSpec((_BLK, _D), lambda i: (i, 0)),
        out_shape=jax.ShapeDtypeStruct((_N, _D), jnp.float32),
    )(acc, y, deg2, b, Wn)


def _tc_final_body(acc_ref, y_ref, deg_ref, b_ref, batch_ref, fw1_ref, fb1_ref,
                   fw2_ref, fb2_ref, out_ref, hg_acc, cnt_acc):
    i = pl.program_id(0)

    @pl.when(i == 0)
    def _():
        hg_acc[...] = jnp.zeros_like(hg_acc)
        cnt_acc[...] = jnp.zeros_like(cnt_acc)

    dinv = _dinv_of(deg_ref)
    s = acc_ref[0] + acc_ref[1] + y_ref[...]
    h = jnp.maximum(dinv * s + b_ref[...], 0.0)
    bvec = batch_ref[0, 0, :]
    onehot = (
        bvec[None, :] == lax.broadcasted_iota(jnp.int32, (_G, _BLK), 0)
    ).astype(jnp.float32)
    hg_acc[...] += jnp.dot(onehot, h, preferred_element_type=jnp.float32)
    cnt_acc[...] += jnp.dot(
        onehot, jnp.ones((_BLK, _D), jnp.float32), preferred_element_type=jnp.float32
    )

    @pl.when(i == _NBLK - 1)
    def _():
        hg = hg_acc[...] / jnp.maximum(cnt_acc[...], 1.0)
        a = jnp.maximum(
            jnp.dot(hg, fw1_ref[...], preferred_element_type=jnp.float32)
            + fb1_ref[...],
            0.0,
        )
        out_ref[...] = (
            jnp.dot(a, fw2_ref[...], preferred_element_type=jnp.float32)
            + fb2_ref[...]
        )


def _tc_final(acc, y, deg2, b, batch3, fW1, fb1, fW2, fb2):
    return pl.pallas_call(
        _tc_final_body,
        grid=(_NBLK,),
        in_specs=[
            pl.BlockSpec((_NC, _BLK, _D), lambda i: (0, i, 0)),
            pl.BlockSpec((_BLK, _D), lambda i: (i, 0)),
            pl.BlockSpec((_NC, _BLK, _D), lambda i: (0, i, 0)),
            pl.BlockSpec((1, _D), lambda i: (0, 0)),
            pl.BlockSpec((1, 1, _BLK), lambda i: (i, 0, 0)),
            pl.BlockSpec((_D, _G), lambda i: (0, 0)),
            pl.BlockSpec((1, _G), lambda i: (0, 0)),
            pl.BlockSpec((_G, 1), lambda i: (0, 0)),
            pl.BlockSpec((1, 1), lambda i: (0, 0)),
        ],
        out_specs=pl.BlockSpec((_G, 1), lambda i: (0, 0)),
        out_shape=jax.ShapeDtypeStruct((_G, 1), jnp.float32),
        scratch_shapes=[
            pltpu.VMEM((_G, _D), jnp.float32),
            pltpu.VMEM((_G, _D), jnp.float32),
        ],
    )(acc, y, deg2, b, batch3, fW1, fb1, fW2, fb2)


def kernel(x, edge_index, batch, W1, b1, W2, b2, W3, b3, fW1, fb1, fW2, fb2):
    src = edge_index[0].astype(jnp.int32)
    dst = edge_index[1].astype(jnp.int32)
    pad_src = jnp.zeros((_E_PAD - _E,), jnp.int32)
    pad_dst = jnp.full((_E_PAD - _E,), _N_PAD - 1, jnp.int32)
    src2d = jnp.concatenate([src, pad_src]).reshape(_E_ROWS, _IDXW)
    dst2d = jnp.concatenate([dst, pad_dst]).reshape(_E_ROWS, _IDXW)

    ones128 = jnp.ones((_IDXW, _D), jnp.float32)
    zeros128 = jnp.zeros((_N_PAD, _D), jnp.float32)

    deg2 = _sc_deg(dst2d, ones128, zeros128)
    y1 = _tc_first(x, W1, deg2)
    acc1 = _sc_edge(y1, src2d, dst2d, zeros128)
    y2 = _tc_mid(acc1, y1, deg2, b1.reshape(1, _D), W2)
    acc2 = _sc_edge(y2, src2d, dst2d, zeros128)
    y3 = _tc_mid(acc2, y2, deg2, b2.reshape(1, _D), W3)
    acc3 = _sc_edge(y3, src2d, dst2d, zeros128)

    batch3 = batch.astype(jnp.int32).reshape(_NBLK, 1, _BLK)
    return _tc_final(
        acc3, y3, deg2, b3.reshape(1, _D), batch3,
        fW1, fb1.reshape(1, _G), fW2, fb2.reshape(1, 1),
    )


# speedup vs baseline: 19.7630x; 3.5513x over previous
import functools

import jax
import jax.numpy as jnp
from jax import lax
from jax.experimental import pallas as pl
from jax.experimental.pallas import tpu as pltpu
from jax.experimental.pallas import tpu_sc as plsc

_N = 10000
_E = 320000
_G = 64
_D = 128

_NC = 2
_NS = 16
_NW = _NC * _NS

_N_PAD = 10240
_IDXW = 128
_E_ROWS = 2560
_E_PAD = _E_ROWS * _IDXW
_IR_PER_TILE = _E_ROWS // _NW

_mesh = plsc.VectorSubcoreMesh(
    core_axis_name="c", subcore_axis_name="s", num_cores=_NC, num_subcores=_NS
)


@functools.partial(
    pl.kernel,
    out_type=jax.ShapeDtypeStruct((_NC, _N_PAD, _D), jnp.float32),
    mesh=_mesh,
    scratch_types=[
        pltpu.VMEM((_IR_PER_TILE, _IDXW), jnp.int32),
        pltpu.VMEM((_IDXW, _D), jnp.float32),
        pltpu.VMEM_SHARED((_N_PAD, _D), jnp.float32),
        pltpu.SemaphoreType.DMA,
    ],
)
def _sc_deg(dst_hbm, ones_hbm, zeros_hbm, out_hbm, dst_v, ones_v, deg_sh, sem):
    cid = lax.axis_index("c")
    sid = lax.axis_index("s")
    wid = cid * _NS + sid
    nz = _N_PAD // _NS
    pltpu.sync_copy(zeros_hbm.at[pl.ds(sid * nz, nz)], deg_sh.at[pl.ds(sid * nz, nz)])
    pltpu.sync_copy(ones_hbm, ones_v)
    pltpu.sync_copy(dst_hbm.at[pl.ds(wid * _IR_PER_TILE, _IR_PER_TILE)], dst_v)
    plsc.subcore_barrier()

    def body(j, carry):
        pltpu.sync_copy(ones_v, deg_sh.at[dst_v.at[j]], add=True)
        return carry

    lax.fori_loop(0, _IR_PER_TILE, body, 0)
    plsc.subcore_barrier()
    pltpu.sync_copy(
        deg_sh.at[pl.ds(sid * nz, nz)], out_hbm.at[cid, pl.ds(sid * nz, nz)]
    )


@functools.partial(
    pl.kernel,
    out_type=jax.ShapeDtypeStruct((_NC, _N_PAD, _D), jnp.float32),
    mesh=_mesh,
    scratch_types=[
        pltpu.VMEM((_IDXW,), jnp.int32),
        pltpu.VMEM((_IDXW,), jnp.int32),
        pltpu.VMEM((_IDXW,), jnp.int32),
        pltpu.VMEM((_IDXW,), jnp.int32),
        pltpu.VMEM((_IDXW, _D), jnp.float32),
        pltpu.VMEM((_IDXW, _D), jnp.float32),
        pltpu.VMEM_SHARED((_N_PAD, _D), jnp.float32),
        pltpu.SemaphoreType.DMA,
        pltpu.SemaphoreType.DMA,
    ],
)
def _sc_edge(y_hbm, src_hbm, dst_hbm, zeros_hbm, out_hbm, src0_v, src1_v,
             dst0_v, dst1_v, rows0_v, rows1_v, acc_sh, sem0, sem1):
    cid = lax.axis_index("c")
    sid = lax.axis_index("s")
    wid = cid * _NS + sid
    nz = _N_PAD // _NS
    base = wid * _IR_PER_TILE
    pltpu.sync_copy(zeros_hbm.at[pl.ds(sid * nz, nz)], acc_sh.at[pl.ds(sid * nz, nz)])
    plsc.subcore_barrier()

    pltpu.sync_copy(src_hbm.at[base], src0_v)
    pltpu.async_copy(y_hbm.at[src0_v], rows0_v, sem0)
    pltpu.sync_copy(dst_hbm.at[base], dst0_v)

    def body(j2, carry):
        ja = j2 * 2
        jb = ja + 1
        pltpu.sync_copy(src_hbm.at[base + jb], src1_v)
        pltpu.async_copy(y_hbm.at[src1_v], rows1_v, sem1)
        pltpu.sync_copy(dst_hbm.at[base + jb], dst1_v)
        pltpu.make_async_copy(y_hbm.at[src0_v], rows0_v, sem0).wait()
        pltpu.sync_copy(rows0_v, acc_sh.at[dst0_v], add=True)

        @pl.when(ja + 2 < _IR_PER_TILE)
        def _():
            pltpu.sync_copy(src_hbm.at[base + ja + 2], src0_v)
            pltpu.async_copy(y_hbm.at[src0_v], rows0_v, sem0)
            pltpu.sync_copy(dst_hbm.at[base + ja + 2], dst0_v)

        pltpu.make_async_copy(y_hbm.at[src1_v], rows1_v, sem1).wait()
        pltpu.sync_copy(rows1_v, acc_sh.at[dst1_v], add=True)
        return carry

    lax.fori_loop(0, _IR_PER_TILE // 2, body, 0)
    plsc.subcore_barrier()
    pltpu.sync_copy(
        acc_sh.at[pl.ds(sid * nz, nz)], out_hbm.at[cid, pl.ds(sid * nz, nz)]
    )


_BLK = 1000
_NBLK = _N // _BLK


def _dinv_of(deg_ref):
    deg = deg_ref[0, :, 0:1] + deg_ref[1, :, 0:1] + 1.0
    return lax.rsqrt(deg)


def _tc_first_body(x_ref, w_ref, deg_ref, y_ref):
    dinv = _dinv_of(deg_ref)
    y_ref[...] = (
        jnp.dot(x_ref[...], w_ref[...], preferred_element_type=jnp.float32) * dinv
    )


def _tc_first(x, W, deg2):
    return pl.pallas_call(
        _tc_first_body,
        grid=(_NBLK,),
        in_specs=[
            pl.BlockSpec((_BLK, _D), lambda i: (i, 0)),
            pl.BlockSpec((_D, _D), lambda i: (0, 0)),
            pl.BlockSpec((_NC, _BLK, _D), lambda i: (0, i, 0)),
        ],
        out_specs=pl.BlockSpec((_BLK, _D), lambda i: (i, 0)),
        out_shape=jax.ShapeDtypeStruct((_N, _D), jnp.float32),
    )(x, W, deg2)


def _tc_mid_body(acc_ref, y_ref, deg_ref, b_ref, w_ref, out_ref):
    dinv = _dinv_of(deg_ref)
    s = acc_ref[0] + acc_ref[1] + y_ref[...]
    h = jnp.maximum(dinv * s + b_ref[...], 0.0)
    out_ref[...] = (
        jnp.dot(h, w_ref[...], preferred_element_type=jnp.float32) * dinv
    )


def _tc_mid(acc, y, deg2, b, Wn):
    return pl.pallas_call(
        _tc_mid_body,
        grid=(_NBLK,),
        in_specs=[
            pl.BlockSpec((_NC, _BLK, _D), lambda i: (0, i, 0)),
            pl.BlockSpec((_BLK, _D), lambda i: (i, 0)),
            pl.BlockSpec((_NC, _BLK, _D), lambda i: (0, i, 0)),
            pl.BlockSpec((1, _D), lambda i: (0, 0)),
            pl.BlockSpec((_D, _D), lambda i: (0, 0)),
        ],
        out_specs=pl.BlockSpec((_BLK, _D), lambda i: (i, 0)),
        out_shape=jax.ShapeDtypeStruct((_N, _D), jnp.float32),
    )(acc, y, deg2, b, Wn)


def _tc_final_body(acc_ref, y_ref, deg_ref, b_ref, batch_ref, fw1_ref, fb1_ref,
                   fw2_ref, fb2_ref, out_ref, hg_acc, cnt_acc):
    i = pl.program_id(0)

    @pl.when(i == 0)
    def _():
        hg_acc[...] = jnp.zeros_like(hg_acc)
        cnt_acc[...] = jnp.zeros_like(cnt_acc)

    dinv = _dinv_of(deg_ref)
    s = acc_ref[0] + acc_ref[1] + y_ref[...]
    h = jnp.maximum(dinv * s + b_ref[...], 0.0)
    bvec = batch_ref[0, 0, :]
    onehot = (
        bvec[None, :] == lax.broadcasted_iota(jnp.int32, (_G, _BLK), 0)
    ).astype(jnp.float32)
    hg_acc[...] += jnp.dot(onehot, h, preferred_element_type=jnp.float32)
    cnt_acc[...] += jnp.dot(
        onehot, jnp.ones((_BLK, _D), jnp.float32), preferred_element_type=jnp.float32
    )

    @pl.when(i == _NBLK - 1)
    def _():
        hg = hg_acc[...] / jnp.maximum(cnt_acc[...], 1.0)
        a = jnp.maximum(
            jnp.dot(hg, fw1_ref[...], preferred_element_type=jnp.float32)
            + fb1_ref[...],
            0.0,
        )
        out_ref[...] = (
            jnp.dot(a, fw2_ref[...], preferred_element_type=jnp.float32)
            + fb2_ref[...]
        )


def _tc_final(acc, y, deg2, b, batch3, fW1, fb1, fW2, fb2):
    return pl.pallas_call(
        _tc_final_body,
        grid=(_NBLK,),
        in_specs=[
            pl.BlockSpec((_NC, _BLK, _D), lambda i: (0, i, 0)),
            pl.BlockSpec((_BLK, _D), lambda i: (i, 0)),
            pl.BlockSpec((_NC, _BLK, _D), lambda i: (0, i, 0)),
            pl.BlockSpec((1, _D), lambda i: (0, 0)),
            pl.BlockSpec((1, 1, _BLK), lambda i: (i, 0, 0)),
            pl.BlockSpec((_D, _G), lambda i: (0, 0)),
            pl.BlockSpec((1, _G), lambda i: (0, 0)),
            pl.BlockSpec((_G, 1), lambda i: (0, 0)),
            pl.BlockSpec((1, 1), lambda i: (0, 0)),
        ],
        out_specs=pl.BlockSpec((_G, 1), lambda i: (0, 0)),
        out_shape=jax.ShapeDtypeStruct((_G, 1), jnp.float32),
        scratch_shapes=[
            pltpu.VMEM((_G, _D), jnp.float32),
            pltpu.VMEM((_G, _D), jnp.float32),
        ],
    )(acc, y, deg2, b, batch3, fW1, fb1, fW2, fb2)


def kernel(x, edge_index, batch, W1, b1, W2, b2, W3, b3, fW1, fb1, fW2, fb2):
    src = edge_index[0].astype(jnp.int32)
    dst = edge_index[1].astype(jnp.int32)
    npad = _E_PAD - _E
    pad_iota = jnp.arange(npad, dtype=jnp.int32)
    pad_src = pad_iota % _N
    pad_dst = _N + pad_iota % (_N_PAD - _N)
    src2d = jnp.concatenate([src, pad_src]).reshape(_E_ROWS, _IDXW)
    dst2d = jnp.concatenate([dst, pad_dst]).reshape(_E_ROWS, _IDXW)

    ones128 = jnp.ones((_IDXW, _D), jnp.float32)
    zeros128 = jnp.zeros((_N_PAD, _D), jnp.float32)

    deg2 = _sc_deg(dst2d, ones128, zeros128)
    y1 = _tc_first(x, W1, deg2)
    acc1 = _sc_edge(y1, src2d, dst2d, zeros128)
    y2 = _tc_mid(acc1, y1, deg2, b1.reshape(1, _D), W2)
    acc2 = _sc_edge(y2, src2d, dst2d, zeros128)
    y3 = _tc_mid(acc2, y2, deg2, b2.reshape(1, _D), W3)
    acc3 = _sc_edge(y3, src2d, dst2d, zeros128)

    batch3 = batch.astype(jnp.int32).reshape(_NBLK, 1, _BLK)
    return _tc_final(
        acc3, y3, deg2, b3.reshape(1, _D), batch3,
        fW1, fb1.reshape(1, _G), fW2, fb2.reshape(1, 1),
    )


# speedup vs baseline: 19.9234x; 1.0081x over previous
import functools

import jax
import jax.numpy as jnp
from jax import lax
from jax.experimental import pallas as pl
from jax.experimental.pallas import tpu as pltpu
from jax.experimental.pallas import tpu_sc as plsc

_N = 10000
_E = 320000
_G = 64
_D = 128

_NC = 2
_NS = 16
_NW = _NC * _NS

_N_PAD = 10240
_IDXW = 128
_E_ROWS = 2560
_E_PAD = _E_ROWS * _IDXW
_IR_PER_TILE = _E_ROWS // _NW

_mesh = plsc.VectorSubcoreMesh(
    core_axis_name="c", subcore_axis_name="s", num_cores=_NC, num_subcores=_NS
)


@functools.partial(
    pl.kernel,
    out_type=jax.ShapeDtypeStruct((_NC, _N_PAD, _D), jnp.float32),
    mesh=_mesh,
    scratch_types=[
        pltpu.VMEM((_IR_PER_TILE, _IDXW), jnp.int32),
        pltpu.VMEM((_IDXW, _D), jnp.float32),
        pltpu.VMEM_SHARED((_N_PAD, _D), jnp.float32),
        pltpu.SemaphoreType.DMA,
    ],
)
def _sc_deg(dst_hbm, ones_hbm, zeros_hbm, out_hbm, dst_v, ones_v, deg_sh, sem):
    cid = lax.axis_index("c")
    sid = lax.axis_index("s")
    wid = cid * _NS + sid
    nz = _N_PAD // _NS
    pltpu.sync_copy(zeros_hbm.at[pl.ds(sid * nz, nz)], deg_sh.at[pl.ds(sid * nz, nz)])
    pltpu.sync_copy(ones_hbm, ones_v)
    pltpu.sync_copy(dst_hbm.at[pl.ds(wid * _IR_PER_TILE, _IR_PER_TILE)], dst_v)
    plsc.subcore_barrier()

    def body(g, carry):
        for k in range(8):
            pltpu.async_copy(ones_v, deg_sh.at[dst_v.at[g * 8 + k]], sem,
                             add=True)
        for k in range(8):
            pltpu.make_async_copy(ones_v, deg_sh.at[dst_v.at[g * 8 + k]],
                                  sem).wait()
        return carry

    lax.fori_loop(0, _IR_PER_TILE // 8, body, 0)
    plsc.subcore_barrier()
    pltpu.sync_copy(
        deg_sh.at[pl.ds(sid * nz, nz)], out_hbm.at[cid, pl.ds(sid * nz, nz)]
    )


@functools.partial(
    pl.kernel,
    out_type=jax.ShapeDtypeStruct((_NC, _N_PAD, _D), jnp.float32),
    mesh=_mesh,
    scratch_types=[
        pltpu.VMEM((2, _IDXW), jnp.int32),
        pltpu.VMEM((2, _IDXW), jnp.int32),
        pltpu.VMEM((2, _IDXW), jnp.int32),
        pltpu.VMEM((2, _IDXW), jnp.int32),
        pltpu.VMEM((_IDXW, _D), jnp.float32),
        pltpu.VMEM((_IDXW, _D), jnp.float32),
        pltpu.VMEM_SHARED((_N_PAD, _D), jnp.float32),
        pltpu.SemaphoreType.DMA,
        pltpu.SemaphoreType.DMA,
        pltpu.SemaphoreType.DMA,
        pltpu.SemaphoreType.DMA,
        pltpu.SemaphoreType.DMA,
    ],
)
def _sc_edge(y_hbm, sd_hbm, zeros_hbm, out_hbm, idx0, idx1, idx2, idx3,
             rows0_v, rows1_v, acc_sh, sem_g0, sem_g1, sem_s0, sem_s1, sem_i):
    cid = lax.axis_index("c")
    sid = lax.axis_index("s")
    wid = cid * _NS + sid
    nz = _N_PAD // _NS
    base = wid * _IR_PER_TILE
    nq = _IR_PER_TILE // 4
    pltpu.sync_copy(zeros_hbm.at[pl.ds(sid * nz, nz)], acc_sh.at[pl.ds(sid * nz, nz)])
    plsc.subcore_barrier()

    pltpu.sync_copy(sd_hbm.at[base], idx0)
    pltpu.sync_copy(sd_hbm.at[base + 1], idx1)
    pltpu.async_copy(y_hbm.at[idx0.at[0]], rows0_v, sem_g0)
    pltpu.async_copy(y_hbm.at[idx1.at[0]], rows1_v, sem_g1)
    pltpu.async_copy(sd_hbm.at[base + 2], idx2, sem_i)
    pltpu.async_copy(sd_hbm.at[base + 3], idx3, sem_i)

    def body(q, carry):
        j0 = base + q * 4
        pltpu.make_async_copy(y_hbm.at[idx0.at[0]], rows0_v, sem_g0).wait()
        pltpu.async_copy(rows0_v, acc_sh.at[idx0.at[1]], sem_s0, add=True)
        pltpu.make_async_copy(y_hbm.at[idx1.at[0]], rows1_v, sem_g1).wait()
        pltpu.async_copy(rows1_v, acc_sh.at[idx1.at[1]], sem_s1, add=True)
        pltpu.make_async_copy(sd_hbm.at[j0 + 2], idx2, sem_i).wait()
        pltpu.make_async_copy(sd_hbm.at[j0 + 3], idx3, sem_i).wait()
        pltpu.make_async_copy(rows0_v, acc_sh.at[idx0.at[1]], sem_s0).wait()
        pltpu.async_copy(y_hbm.at[idx2.at[0]], rows0_v, sem_g0)
        pltpu.make_async_copy(rows1_v, acc_sh.at[idx1.at[1]], sem_s1).wait()
        pltpu.async_copy(y_hbm.at[idx3.at[0]], rows1_v, sem_g1)

        @pl.when(q < nq - 1)
        def _():
            pltpu.async_copy(sd_hbm.at[j0 + 4], idx0, sem_i)
            pltpu.async_copy(sd_hbm.at[j0 + 5], idx1, sem_i)

        pltpu.make_async_copy(y_hbm.at[idx2.at[0]], rows0_v, sem_g0).wait()
        pltpu.async_copy(rows0_v, acc_sh.at[idx2.at[1]], sem_s0, add=True)
        pltpu.make_async_copy(y_hbm.at[idx3.at[0]], rows1_v, sem_g1).wait()
        pltpu.async_copy(rows1_v, acc_sh.at[idx3.at[1]], sem_s1, add=True)

        @pl.when(q < nq - 1)
        def _():
            pltpu.make_async_copy(sd_hbm.at[j0 + 4], idx0, sem_i).wait()
            pltpu.make_async_copy(sd_hbm.at[j0 + 5], idx1, sem_i).wait()
            pltpu.make_async_copy(rows0_v, acc_sh.at[idx2.at[1]], sem_s0).wait()
            pltpu.async_copy(y_hbm.at[idx0.at[0]], rows0_v, sem_g0)
            pltpu.make_async_copy(rows1_v, acc_sh.at[idx3.at[1]], sem_s1).wait()
            pltpu.async_copy(y_hbm.at[idx1.at[0]], rows1_v, sem_g1)
            pltpu.async_copy(sd_hbm.at[j0 + 6], idx2, sem_i)
            pltpu.async_copy(sd_hbm.at[j0 + 7], idx3, sem_i)

        return carry

    lax.fori_loop(0, nq, body, 0)
    pltpu.make_async_copy(rows0_v, acc_sh.at[idx2.at[1]], sem_s0).wait()
    pltpu.make_async_copy(rows1_v, acc_sh.at[idx3.at[1]], sem_s1).wait()
    plsc.subcore_barrier()
    pltpu.sync_copy(
        acc_sh.at[pl.ds(sid * nz, nz)], out_hbm.at[cid, pl.ds(sid * nz, nz)]
    )


_BLK = 1000
_NBLK = _N // _BLK


def _dinv_of(deg_ref):
    deg = deg_ref[0, :, 0:1] + deg_ref[1, :, 0:1] + 1.0
    return lax.rsqrt(deg)


def _tc_first_body(x_ref, w_ref, deg_ref, y_ref):
    dinv = _dinv_of(deg_ref)
    y_ref[...] = (
        jnp.dot(x_ref[...], w_ref[...], preferred_element_type=jnp.float32) * dinv
    )


def _tc_first(x, W, deg2):
    return pl.pallas_call(
        _tc_first_body,
        grid=(_NBLK,),
        in_specs=[
            pl.BlockSpec((_BLK, _D), lambda i: (i, 0)),
            pl.BlockSpec((_D, _D), lambda i: (0, 0)),
            pl.BlockSpec((_NC, _BLK, _D), lambda i: (0, i, 0)),
        ],
        out_specs=pl.BlockSpec((_BLK, _D), lambda i: (i, 0)),
        out_shape=jax.ShapeDtypeStruct((_N, _D), jnp.float32),
    )(x, W, deg2)


def _tc_mid_body(acc_ref, y_ref, deg_ref, b_ref, w_ref, out_ref):
    dinv = _dinv_of(deg_ref)
    s = acc_ref[0] + acc_ref[1] + y_ref[...]
    h = jnp.maximum(dinv * s + b_ref[...], 0.0)
    out_ref[...] = (
        jnp.dot(h, w_ref[...], preferred_element_type=jnp.float32) * dinv
    )


def _tc_mid(acc, y, deg2, b, Wn):
    return pl.pallas_call(
        _tc_mid_body,
        grid=(_NBLK,),
        in_specs=[
            pl.BlockSpec((_NC, _BLK, _D), lambda i: (0, i, 0)),
            pl.BlockSpec((_BLK, _D), lambda i: (i, 0)),
            pl.BlockSpec((_NC, _BLK, _D), lambda i: (0, i, 0)),
            pl.BlockSpec((1, _D), lambda i: (0, 0)),
            pl.BlockSpec((_D, _D), lambda i: (0, 0)),
        ],
        out_specs=pl.BlockSpec((_BLK, _D), lambda i: (i, 0)),
        out_shape=jax.ShapeDtypeStruct((_N, _D), jnp.float32),
    )(acc, y, deg2, b, Wn)


def _tc_final_body(acc_ref, y_ref, deg_ref, b_ref, batch_ref, fw1_ref, fb1_ref,
                   fw2_ref, fb2_ref, out_ref, hg_acc, cnt_acc):
    i = pl.program_id(0)

    @pl.when(i == 0)
    def _():
        hg_acc[...] = jnp.zeros_like(hg_acc)
        cnt_acc[...] = jnp.zeros_like(cnt_acc)

    dinv = _dinv_of(deg_ref)
    s = acc_ref[0] + acc_ref[1] + y_ref[...]
    h = jnp.maximum(dinv * s + b_ref[...], 0.0)
    bvec = batch_ref[0, 0, :]
    onehot = (
        bvec[None, :] == lax.broadcasted_iota(jnp.int32, (_G, _BLK), 0)
    ).astype(jnp.float32)
    hg_acc[...] += jnp.dot(onehot, h, preferred_element_type=jnp.float32)
    cnt_acc[...] += jnp.dot(
        onehot, jnp.ones((_BLK, _D), jnp.float32), preferred_element_type=jnp.float32
    )

    @pl.when(i == _NBLK - 1)
    def _():
        hg = hg_acc[...] / jnp.maximum(cnt_acc[...], 1.0)
        a = jnp.maximum(
            jnp.dot(hg, fw1_ref[...], preferred_element_type=jnp.float32)
            + fb1_ref[...],
            0.0,
        )
        out_ref[...] = (
            jnp.dot(a, fw2_ref[...], preferred_element_type=jnp.float32)
            + fb2_ref[...]
        )


def _tc_final(acc, y, deg2, b, batch3, fW1, fb1, fW2, fb2):
    return pl.pallas_call(
        _tc_final_body,
        grid=(_NBLK,),
        in_specs=[
            pl.BlockSpec((_NC, _BLK, _D), lambda i: (0, i, 0)),
            pl.BlockSpec((_BLK, _D), lambda i: (i, 0)),
            pl.BlockSpec((_NC, _BLK, _D), lambda i: (0, i, 0)),
            pl.BlockSpec((1, _D), lambda i: (0, 0)),
            pl.BlockSpec((1, 1, _BLK), lambda i: (i, 0, 0)),
            pl.BlockSpec((_D, _G), lambda i: (0, 0)),
            pl.BlockSpec((1, _G), lambda i: (0, 0)),
            pl.BlockSpec((_G, 1), lambda i: (0, 0)),
            pl.BlockSpec((1, 1), lambda i: (0, 0)),
        ],
        out_specs=pl.BlockSpec((_G, 1), lambda i: (0, 0)),
        out_shape=jax.ShapeDtypeStruct((_G, 1), jnp.float32),
        scratch_shapes=[
            pltpu.VMEM((_G, _D), jnp.float32),
            pltpu.VMEM((_G, _D), jnp.float32),
        ],
    )(acc, y, deg2, b, batch3, fW1, fb1, fW2, fb2)


def kernel(x, edge_index, batch, W1, b1, W2, b2, W3, b3, fW1, fb1, fW2, fb2):
    src = edge_index[0].astype(jnp.int32)
    dst = edge_index[1].astype(jnp.int32)
    npad = _E_PAD - _E
    pad_iota = jnp.arange(npad, dtype=jnp.int32)
    pad_src = pad_iota % _N
    pad_dst = _N + pad_iota % (_N_PAD - _N)
    src2d = jnp.concatenate([src, pad_src]).reshape(_E_ROWS, _IDXW)
    dst2d = jnp.concatenate([dst, pad_dst]).reshape(_E_ROWS, _IDXW)
    sd2d = jnp.stack([src2d, dst2d], axis=1)

    ones128 = jnp.ones((_IDXW, _D), jnp.float32)
    zeros128 = jnp.zeros((_N_PAD, _D), jnp.float32)

    deg2 = _sc_deg(dst2d, ones128, zeros128)
    y1 = _tc_first(x, W1, deg2)
    acc1 = _sc_edge(y1, sd2d, zeros128)
    y2 = _tc_mid(acc1, y1, deg2, b1.reshape(1, _D), W2)
    acc2 = _sc_edge(y2, sd2d, zeros128)
    y3 = _tc_mid(acc2, y2, deg2, b2.reshape(1, _D), W3)
    acc3 = _sc_edge(y3, sd2d, zeros128)

    batch3 = batch.astype(jnp.int32).reshape(_NBLK, 1, _BLK)
    return _tc_final(
        acc3, y3, deg2, b3.reshape(1, _D), batch3,
        fW1, fb1.reshape(1, _G), fW2, fb2.reshape(1, 1),
    )


# speedup vs baseline: 20.0272x; 1.0052x over previous
import functools

import jax
import jax.numpy as jnp
from jax import lax
from jax.experimental import pallas as pl
from jax.experimental.pallas import tpu as pltpu
from jax.experimental.pallas import tpu_sc as plsc

_N = 10000
_E = 320000
_G = 64
_D = 128

_NC = 2
_NS = 16
_NW = _NC * _NS

_N_PAD = 10240
_IDXW = 128
_E_ROWS = 2560
_E_PAD = _E_ROWS * _IDXW
_IR_PER_TILE = _E_ROWS // _NW

_mesh = plsc.VectorSubcoreMesh(
    core_axis_name="c", subcore_axis_name="s", num_cores=_NC, num_subcores=_NS
)


@functools.partial(
    pl.kernel,
    out_type=jax.ShapeDtypeStruct((_NC, _N_PAD, _D), jnp.float32),
    mesh=_mesh,
    scratch_types=[
        pltpu.VMEM((_IR_PER_TILE, _IDXW), jnp.int32),
        pltpu.VMEM((_IDXW, _D), jnp.float32),
        pltpu.VMEM_SHARED((_N_PAD, _D), jnp.float32),
        pltpu.SemaphoreType.DMA,
    ],
)
def _sc_deg(dst_hbm, ones_hbm, zeros_hbm, out_hbm, dst_v, ones_v, deg_sh, sem):
    cid = lax.axis_index("c")
    sid = lax.axis_index("s")
    wid = cid * _NS + sid
    nz = _N_PAD // _NS
    pltpu.sync_copy(zeros_hbm.at[pl.ds(sid * nz, nz)], deg_sh.at[pl.ds(sid * nz, nz)])
    pltpu.sync_copy(ones_hbm, ones_v)
    pltpu.sync_copy(dst_hbm.at[pl.ds(wid * _IR_PER_TILE, _IR_PER_TILE)], dst_v)
    plsc.subcore_barrier()

    def body(g, carry):
        for k in range(8):
            pltpu.async_copy(ones_v, deg_sh.at[dst_v.at[g * 8 + k]], sem,
                             add=True)
        for k in range(8):
            pltpu.make_async_copy(ones_v, deg_sh.at[dst_v.at[g * 8 + k]],
                                  sem).wait()
        return carry

    lax.fori_loop(0, _IR_PER_TILE // 8, body, 0)
    plsc.subcore_barrier()
    pltpu.sync_copy(
        deg_sh.at[pl.ds(sid * nz, nz)], out_hbm.at[cid, pl.ds(sid * nz, nz)]
    )


@functools.partial(
    pl.kernel,
    out_type=jax.ShapeDtypeStruct((_NC, _N_PAD, _D), jnp.float32),
    mesh=_mesh,
    scratch_types=[
        pltpu.VMEM((2, _IDXW), jnp.int32),
        pltpu.VMEM((2, _IDXW), jnp.int32),
        pltpu.VMEM((2, _IDXW), jnp.int32),
        pltpu.VMEM((2, _IDXW), jnp.int32),
        pltpu.VMEM((_IDXW, _D), jnp.float32),
        pltpu.VMEM((_IDXW, _D), jnp.float32),
        pltpu.VMEM_SHARED((_N_PAD, _D), jnp.float32),
        pltpu.SemaphoreType.DMA,
        pltpu.SemaphoreType.DMA,
        pltpu.SemaphoreType.DMA,
        pltpu.SemaphoreType.DMA,
        pltpu.SemaphoreType.DMA,
    ],
)
def _sc_edge(y_hbm, sd_hbm, zeros_hbm, out_hbm, idx0, idx1, idx2, idx3,
             rows0_v, rows1_v, acc_sh, sem_g0, sem_g1, sem_s0, sem_s1, sem_i):
    cid = lax.axis_index("c")
    sid = lax.axis_index("s")
    wid = cid * _NS + sid
    nz = _N_PAD // _NS
    base = wid * _IR_PER_TILE
    nq = _IR_PER_TILE // 4
    pltpu.sync_copy(zeros_hbm.at[pl.ds(sid * nz, nz)], acc_sh.at[pl.ds(sid * nz, nz)])
    plsc.subcore_barrier()

    pltpu.sync_copy(sd_hbm.at[base], idx0)
    pltpu.sync_copy(sd_hbm.at[base + 1], idx1)
    pltpu.async_copy(y_hbm.at[idx0.at[0]], rows0_v, sem_g0)
    pltpu.async_copy(y_hbm.at[idx1.at[0]], rows1_v, sem_g1)
    pltpu.async_copy(sd_hbm.at[base + 2], idx2, sem_i)
    pltpu.async_copy(sd_hbm.at[base + 3], idx3, sem_i)

    def body(q, carry):
        j0 = base + q * 4
        pltpu.make_async_copy(y_hbm.at[idx0.at[0]], rows0_v, sem_g0).wait()
        pltpu.async_copy(rows0_v, acc_sh.at[idx0.at[1]], sem_s0, add=True)
        pltpu.make_async_copy(y_hbm.at[idx1.at[0]], rows1_v, sem_g1).wait()
        pltpu.async_copy(rows1_v, acc_sh.at[idx1.at[1]], sem_s1, add=True)
        pltpu.make_async_copy(sd_hbm.at[j0 + 2], idx2, sem_i).wait()
        pltpu.make_async_copy(sd_hbm.at[j0 + 3], idx3, sem_i).wait()
        pltpu.make_async_copy(rows0_v, acc_sh.at[idx0.at[1]], sem_s0).wait()
        pltpu.async_copy(y_hbm.at[idx2.at[0]], rows0_v, sem_g0)
        pltpu.make_async_copy(rows1_v, acc_sh.at[idx1.at[1]], sem_s1).wait()
        pltpu.async_copy(y_hbm.at[idx3.at[0]], rows1_v, sem_g1)

        @pl.when(q < nq - 1)
        def _():
            pltpu.async_copy(sd_hbm.at[j0 + 4], idx0, sem_i)
            pltpu.async_copy(sd_hbm.at[j0 + 5], idx1, sem_i)

        pltpu.make_async_copy(y_hbm.at[idx2.at[0]], rows0_v, sem_g0).wait()
        pltpu.async_copy(rows0_v, acc_sh.at[idx2.at[1]], sem_s0, add=True)
        pltpu.make_async_copy(y_hbm.at[idx3.at[0]], rows1_v, sem_g1).wait()
        pltpu.async_copy(rows1_v, acc_sh.at[idx3.at[1]], sem_s1, add=True)

        @pl.when(q < nq - 1)
        def _():
            pltpu.make_async_copy(sd_hbm.at[j0 + 4], idx0, sem_i).wait()
            pltpu.make_async_copy(sd_hbm.at[j0 + 5], idx1, sem_i).wait()
            pltpu.make_async_copy(rows0_v, acc_sh.at[idx2.at[1]], sem_s0).wait()
            pltpu.async_copy(y_hbm.at[idx0.at[0]], rows0_v, sem_g0)
            pltpu.make_async_copy(rows1_v, acc_sh.at[idx3.at[1]], sem_s1).wait()
            pltpu.async_copy(y_hbm.at[idx1.at[0]], rows1_v, sem_g1)
            pltpu.async_copy(sd_hbm.at[j0 + 6], idx2, sem_i)
            pltpu.async_copy(sd_hbm.at[j0 + 7], idx3, sem_i)

        return carry

    lax.fori_loop(0, nq, body, 0)
    pltpu.make_async_copy(rows0_v, acc_sh.at[idx2.at[1]], sem_s0).wait()
    pltpu.make_async_copy(rows1_v, acc_sh.at[idx3.at[1]], sem_s1).wait()
    plsc.subcore_barrier()
    pltpu.sync_copy(
        acc_sh.at[pl.ds(sid * nz, nz)], out_hbm.at[cid, pl.ds(sid * nz, nz)]
    )


_BLK = 1000
_NBLK = _N // _BLK


def _dinv_of(deg_ref):
    deg = deg_ref[0, :, 0:1] + deg_ref[1, :, 0:1] + 1.0
    return lax.rsqrt(deg)


def _tc_first_body(x_ref, w_ref, deg_ref, y_ref, dinv_ref):
    dinv = _dinv_of(deg_ref)
    y_ref[...] = (
        jnp.dot(x_ref[...], w_ref[...], preferred_element_type=jnp.float32) * dinv
    )
    dinv_ref[0, 0, :] = dinv[:, 0]


def _tc_first(x, W, deg2):
    return pl.pallas_call(
        _tc_first_body,
        grid=(_NBLK,),
        in_specs=[
            pl.BlockSpec((_BLK, _D), lambda i: (i, 0)),
            pl.BlockSpec((_D, _D), lambda i: (0, 0)),
            pl.BlockSpec((_NC, _BLK, _D), lambda i: (0, i, 0)),
        ],
        out_specs=[
            pl.BlockSpec((_BLK, _D), lambda i: (i, 0)),
            pl.BlockSpec((1, 1, _BLK), lambda i: (i, 0, 0)),
        ],
        out_shape=[
            jax.ShapeDtypeStruct((_N, _D), jnp.float32),
            jax.ShapeDtypeStruct((_NBLK, 1, _BLK), jnp.float32),
        ],
    )(x, W, deg2)


def _tc_mid_body(acc_ref, y_ref, dinv_ref, b_ref, w_ref, out_ref):
    dinv = dinv_ref[0, 0, :].reshape(_BLK, 1)
    s = acc_ref[0] + acc_ref[1] + y_ref[...]
    h = jnp.maximum(dinv * s + b_ref[...], 0.0)
    out_ref[...] = (
        jnp.dot(h, w_ref[...], preferred_element_type=jnp.float32) * dinv
    )


def _tc_mid(acc, y, dinv3, b, Wn):
    return pl.pallas_call(
        _tc_mid_body,
        grid=(_NBLK,),
        in_specs=[
            pl.BlockSpec((_NC, _BLK, _D), lambda i: (0, i, 0)),
            pl.BlockSpec((_BLK, _D), lambda i: (i, 0)),
            pl.BlockSpec((1, 1, _BLK), lambda i: (i, 0, 0)),
            pl.BlockSpec((1, _D), lambda i: (0, 0)),
            pl.BlockSpec((_D, _D), lambda i: (0, 0)),
        ],
        out_specs=pl.BlockSpec((_BLK, _D), lambda i: (i, 0)),
        out_shape=jax.ShapeDtypeStruct((_N, _D), jnp.float32),
    )(acc, y, dinv3, b, Wn)


def _tc_final_body(acc_ref, y_ref, dinv_ref, b_ref, batch_ref, fw1_ref, fb1_ref,
                   fw2_ref, fb2_ref, out_ref, hg_acc, cnt_acc):
    i = pl.program_id(0)

    @pl.when(i == 0)
    def _():
        hg_acc[...] = jnp.zeros_like(hg_acc)
        cnt_acc[...] = jnp.zeros_like(cnt_acc)

    dinv = dinv_ref[0, 0, :].reshape(_BLK, 1)
    s = acc_ref[0] + acc_ref[1] + y_ref[...]
    h = jnp.maximum(dinv * s + b_ref[...], 0.0)
    bvec = batch_ref[0, 0, :]
    onehot = (
        bvec[None, :] == lax.broadcasted_iota(jnp.int32, (_G, _BLK), 0)
    ).astype(jnp.float32)
    hg_acc[...] += jnp.dot(onehot, h, preferred_element_type=jnp.float32)
    cnt_acc[...] += jnp.dot(
        onehot, jnp.ones((_BLK, _D), jnp.float32), preferred_element_type=jnp.float32
    )

    @pl.when(i == _NBLK - 1)
    def _():
        hg = hg_acc[...] / jnp.maximum(cnt_acc[...], 1.0)
        a = jnp.maximum(
            jnp.dot(hg, fw1_ref[...], preferred_element_type=jnp.float32)
            + fb1_ref[...],
            0.0,
        )
        out_ref[...] = (
            jnp.dot(a, fw2_ref[...], preferred_element_type=jnp.float32)
            + fb2_ref[...]
        )


def _tc_final(acc, y, dinv3, b, batch3, fW1, fb1, fW2, fb2):
    return pl.pallas_call(
        _tc_final_body,
        grid=(_NBLK,),
        in_specs=[
            pl.BlockSpec((_NC, _BLK, _D), lambda i: (0, i, 0)),
            pl.BlockSpec((_BLK, _D), lambda i: (i, 0)),
            pl.BlockSpec((1, 1, _BLK), lambda i: (i, 0, 0)),
            pl.BlockSpec((1, _D), lambda i: (0, 0)),
            pl.BlockSpec((1, 1, _BLK), lambda i: (i, 0, 0)),
            pl.BlockSpec((_D, _G), lambda i: (0, 0)),
            pl.BlockSpec((1, _G), lambda i: (0, 0)),
            pl.BlockSpec((_G, 1), lambda i: (0, 0)),
            pl.BlockSpec((1, 1), lambda i: (0, 0)),
        ],
        out_specs=pl.BlockSpec((_G, 1), lambda i: (0, 0)),
        out_shape=jax.ShapeDtypeStruct((_G, 1), jnp.float32),
        scratch_shapes=[
            pltpu.VMEM((_G, _D), jnp.float32),
            pltpu.VMEM((_G, _D), jnp.float32),
        ],
    )(acc, y, dinv3, b, batch3, fW1, fb1, fW2, fb2)


def kernel(x, edge_index, batch, W1, b1, W2, b2, W3, b3, fW1, fb1, fW2, fb2):
    src = edge_index[0].astype(jnp.int32)
    dst = edge_index[1].astype(jnp.int32)
    npad = _E_PAD - _E
    pad_iota = jnp.arange(npad, dtype=jnp.int32)
    pad_src = pad_iota % _N
    pad_dst = _N + pad_iota % (_N_PAD - _N)
    src2d = jnp.concatenate([src, pad_src]).reshape(_E_ROWS, _IDXW)
    dst2d = jnp.concatenate([dst, pad_dst]).reshape(_E_ROWS, _IDXW)
    sd2d = jnp.stack([src2d, dst2d], axis=1)

    ones128 = jnp.ones((_IDXW, _D), jnp.float32)
    zeros128 = jnp.zeros((_N_PAD, _D), jnp.float32)

    deg2 = _sc_deg(dst2d, ones128, zeros128)
    y1, dinv3 = _tc_first(x, W1, deg2)
    acc1 = _sc_edge(y1, sd2d, zeros128)
    y2 = _tc_mid(acc1, y1, dinv3, b1.reshape(1, _D), W2)
    acc2 = _sc_edge(y2, sd2d, zeros128)
    y3 = _tc_mid(acc2, y2, dinv3, b2.reshape(1, _D), W3)
    acc3 = _sc_edge(y3, sd2d, zeros128)

    batch3 = batch.astype(jnp.int32).reshape(_NBLK, 1, _BLK)
    return _tc_final(
        acc3, y3, dinv3, b3.reshape(1, _D), batch3,
        fW1, fb1.reshape(1, _G), fW2, fb2.reshape(1, 1),
    )
